# baseline (speedup 1.0000x reference)
import jax
import jax.numpy as jnp
from jax.experimental import pallas as pl
from jax.experimental.pallas import tpu as pltpu
from jax.experimental.pallas import tpu_sc as plsc

B, L, HIDDEN = 2, 2048, 1024
HEADS, HEAD_DIM = 16, 64
CLUSTERS, ITERS = 128, 2

_f32 = jnp.float32
_NIDX = B * HEADS * L
_NROWS = B * HEADS * CLUSTERS
_GW = 256
_TW = 128


def _mm_seq(a, w, chunk=256):
    k = a.shape[1]
    acc = jnp.dot(a[:, 0:chunk], w[0:chunk], preferred_element_type=_f32)
    for i in range(1, k // chunk):
        acc = acc + jnp.dot(a[:, chunk * i:chunk * (i + 1)],
                            w[chunk * i:chunk * (i + 1)],
                            preferred_element_type=_f32)
    return acc


def _mm_t_seq(a, bmat, chunk=256):
    k = a.shape[0]
    dn = (((0,), (0,)), ((), ()))
    acc = jax.lax.dot_general(a[0:chunk], bmat[0:chunk], dn,
                              preferred_element_type=_f32)
    for i in range(1, k // chunk):
        acc = acc + jax.lax.dot_general(a[chunk * i:chunk * (i + 1)],
                                        bmat[chunk * i:chunk * (i + 1)], dn,
                                        preferred_element_type=_f32)
    return acc


def _attn_body(x_ref, xi_ref, maskr_ref, maskc_ref,
               wq_ref, bq_ref, wk_ref, bk_ref, wv_ref, bv_ref,
               outc_ref, gid_ref):
    b = pl.program_id(0)
    h = pl.program_id(1)
    x = x_ref[0]
    q = _mm_seq(x, wq_ref[0]) + bq_ref[0]
    k = _mm_seq(x, wk_ref[0]) + bk_ref[0]
    v = _mm_seq(x, wv_ref[0]) + bv_ref[0]
    maskc = maskc_ref[0]
    maskr = maskr_ref[0]

    cent = _mm_seq(xi_ref[0], wq_ref[0]) + bq_ref[0]
    qsq = jnp.sum(q * q, axis=1, keepdims=True)
    iota_c = jax.lax.broadcasted_iota(jnp.int32, (L, CLUSTERS), 1)
    ones_col = jnp.ones((L, 1), _f32)
    grp = None
    for _ in range(ITERS):
        centsq = jnp.sum(cent * cent, axis=1)
        qc = jax.lax.dot_general(q, cent, (((1,), (1,)), ((), ())),
                                 preferred_element_type=_f32)
        d = qsq - 2.0 * qc + centsq.reshape(1, CLUSTERS)
        dmin = jnp.min(d, axis=1, keepdims=True)
        grp = jnp.min(jnp.where(d == dmin, iota_c, CLUSTERS),
                      axis=1, keepdims=True)
        onehot = jnp.where(iota_c == grp, 1.0, 0.0).astype(_f32) * maskc
        counts = jax.lax.dot_general(onehot, ones_col, (((0,), (0,)), ((), ())),
                                     preferred_element_type=_f32)
        sums = _mm_t_seq(onehot, q)
        new_cent = sums / jnp.maximum(counts, 1.0)
        cent = jnp.where(counts > 0, new_cent, cent)

    scale = _f32(1.0 / (HEAD_DIM ** 0.5))
    logits = jax.lax.dot_general(cent, k, (((1,), (1,)), ((), ())),
                                 preferred_element_type=_f32) * scale
    logits = jnp.where(maskr > 0.0, logits, _f32(-1e9))
    mx = jnp.max(logits, axis=1, keepdims=True)
    e = jnp.exp(logits - mx)
    a = e / jnp.sum(e, axis=1, keepdims=True)
    out_c = jnp.dot(a, v, preferred_element_type=_f32)
    outc_ref[0, 0] = jnp.concatenate(
        [out_c, jnp.zeros((CLUSTERS, _TW - HEAD_DIM), _f32)], axis=1)
    gid_ref[0, 0] = grp + (b * HEADS + h) * CLUSTERS


def _proj_body(g_ref, maskc_ref, wo_ref, bo_ref, y_ref):
    h = pl.program_id(1)
    part = jnp.dot(g_ref[0, 0][:, :HEAD_DIM], wo_ref[0],
                   preferred_element_type=_f32)

    @pl.when(h == 0)
    def _():
        y_ref[0] = part

    @pl.when(h != 0)
    def _():
        y_ref[0] = y_ref[0] + part

    @pl.when(h == HEADS - 1)
    def _():
        y_ref[0] = y_ref[0] * maskc_ref[0] + bo_ref[...]


def _sc_gather(table, idx):
    mesh = plsc.VectorSubcoreMesh(core_axis_name="core",
                                  subcore_axis_name="subcore")

    @pl.kernel(out_type=jax.ShapeDtypeStruct((_NIDX, _TW), _f32),
               mesh=mesh)
    def gather_kernel(tab_hbm, i_hbm, o_hbm):
        def body(i_vmem, o_vmem):
            pltpu.sync_copy(tab_hbm.at[i_vmem.at[0]], o_vmem)

        pltpu.emit_pipeline(
            body,
            grid=(_NIDX // _GW,),
            in_specs=[pl.BlockSpec((1, _GW), index_map=lambda i: (0, i))],
            out_specs=[pl.BlockSpec((_GW, _TW), index_map=lambda i: (i, 0))],
            core_axis_name=("core", "subcore"),
            dimension_semantics=(pltpu.PARALLEL,),
        )(i_hbm, o_hbm)

    return gather_kernel(table, idx)


def kernel(X, attn_mask, length_mask, Wq, bq, Wk, bk, Wv, bv, Wo, bo):
    pos = jnp.arange(L, dtype=jnp.int32)
    maskf = (attn_mask & (pos[None, :] < length_mask[:, None])).astype(_f32)
    maskr = maskf.reshape(B, 1, L)
    maskc = maskf.reshape(B, L, 1)
    init_idx = jnp.linspace(0, L - 1, CLUSTERS).astype(jnp.int32)
    xinit = X[:, init_idx, :]

    wq3 = Wq.reshape(HIDDEN, HEADS, HEAD_DIM).transpose(1, 0, 2)
    wk3 = Wk.reshape(HIDDEN, HEADS, HEAD_DIM).transpose(1, 0, 2)
    wv3 = Wv.reshape(HIDDEN, HEADS, HEAD_DIM).transpose(1, 0, 2)
    wo3 = Wo.reshape(HEADS, HEAD_DIM, HIDDEN)

    out_c, gid = pl.pallas_call(
        _attn_body,
        grid=(B, HEADS),
        in_specs=[
            pl.BlockSpec((1, L, HIDDEN), lambda b, h: (b, 0, 0)),
            pl.BlockSpec((1, CLUSTERS, HIDDEN), lambda b, h: (b, 0, 0)),
            pl.BlockSpec((1, 1, L), lambda b, h: (b, 0, 0)),
            pl.BlockSpec((1, L, 1), lambda b, h: (b, 0, 0)),
            pl.BlockSpec((1, HIDDEN, HEAD_DIM), lambda b, h: (h, 0, 0)),
            pl.BlockSpec((1, 1, HEAD_DIM), lambda b, h: (h, 0, 0)),
            pl.BlockSpec((1, HIDDEN, HEAD_DIM), lambda b, h: (h, 0, 0)),
            pl.BlockSpec((1, 1, HEAD_DIM), lambda b, h: (h, 0, 0)),
            pl.BlockSpec((1, HIDDEN, HEAD_DIM), lambda b, h: (h, 0, 0)),
            pl.BlockSpec((1, 1, HEAD_DIM), lambda b, h: (h, 0, 0)),
        ],
        out_specs=[
            pl.BlockSpec((1, 1, CLUSTERS, _TW), lambda b, h: (b, h, 0, 0)),
            pl.BlockSpec((1, 1, L, 1), lambda b, h: (b, h, 0, 0)),
        ],
        out_shape=[
            jax.ShapeDtypeStruct((B, HEADS, CLUSTERS, _TW), _f32),
            jax.ShapeDtypeStruct((B, HEADS, L, 1), jnp.int32),
        ],
    )(X, xinit, maskr, maskc,
      wq3, bq.reshape(HEADS, 1, HEAD_DIM), wk3, bk.reshape(HEADS, 1, HEAD_DIM),
      wv3, bv.reshape(HEADS, 1, HEAD_DIM))

    table = out_c.reshape(_NROWS, _TW)
    idx = gid.reshape(1, _NIDX)
    gath = _sc_gather(table, idx).reshape(B, HEADS, L, _TW)

    out = pl.pallas_call(
        _proj_body,
        grid=(B, HEADS),
        in_specs=[
            pl.BlockSpec((1, 1, L, _TW), lambda b, h: (b, h, 0, 0)),
            pl.BlockSpec((1, L, 1), lambda b, h: (b, 0, 0)),
            pl.BlockSpec((1, HEAD_DIM, HIDDEN), lambda b, h: (h, 0, 0)),
            pl.BlockSpec((1, HIDDEN), lambda b, h: (0, 0)),
        ],
        out_specs=pl.BlockSpec((1, L, HIDDEN), lambda b, h: (b, 0, 0)),
        out_shape=jax.ShapeDtypeStruct((B, L, HIDDEN), _f32),
    )(gath, maskc, wo3, bo.reshape(1, -1))
    return out


# speedup vs baseline: 1.0506x; 1.0506x over previous
import jax
import jax.numpy as jnp
from jax.experimental import pallas as pl
from jax.experimental.pallas import tpu as pltpu
from jax.experimental.pallas import tpu_sc as plsc

B, L, HIDDEN = 2, 2048, 1024
HEADS, HEAD_DIM = 16, 64
CLUSTERS, ITERS = 128, 2

_f32 = jnp.float32
_NIDX = B * HEADS * L
_NROWS = B * HEADS * CLUSTERS
_GW = 256
_TW = 128
_LT = 1024


def _mm_seq(a, w, chunk=256):
    k = a.shape[1]
    acc = jnp.dot(a[:, 0:chunk], w[0:chunk], preferred_element_type=_f32)
    for i in range(1, k // chunk):
        acc = acc + jnp.dot(a[:, chunk * i:chunk * (i + 1)],
                            w[chunk * i:chunk * (i + 1)],
                            preferred_element_type=_f32)
    return acc


def _mm_t_seq(a, bmat, chunk=256):
    k = a.shape[0]
    dn = (((0,), (0,)), ((), ()))
    acc = jax.lax.dot_general(a[0:chunk], bmat[0:chunk], dn,
                              preferred_element_type=_f32)
    for i in range(1, k // chunk):
        acc = acc + jax.lax.dot_general(a[chunk * i:chunk * (i + 1)],
                                        bmat[chunk * i:chunk * (i + 1)], dn,
                                        preferred_element_type=_f32)
    return acc


def _attn_body(x_ref, xi_ref, maskc_ref,
               wqkv_ref, bqkv_ref, outc_ref, gid_ref):
    b = pl.program_id(0)
    h = pl.program_id(1)
    x = x_ref[0]
    w = wqkv_ref[0]
    bias = bqkv_ref[0]
    qkv = _mm_seq(x, w) + bias
    q = qkv[:, 0:HEAD_DIM]
    k = qkv[:, HEAD_DIM:2 * HEAD_DIM]
    v = qkv[:, 2 * HEAD_DIM:3 * HEAD_DIM]
    maskc = maskc_ref[0]

    cent = (_mm_seq(xi_ref[0], w) + bias)[:, 0:HEAD_DIM]
    qsq = jnp.sum(q * q, axis=1, keepdims=True)
    iota_c = jax.lax.broadcasted_iota(jnp.int32, (L, CLUSTERS), 1)
    ones_col = jnp.ones((L, 1), _f32)
    grp = None
    for _ in range(ITERS):
        centsq = jnp.sum(cent * cent, axis=1)
        qc = jax.lax.dot_general(q, cent, (((1,), (1,)), ((), ())),
                                 preferred_element_type=_f32)
        d = qsq - 2.0 * qc + centsq.reshape(1, CLUSTERS)
        dmin = jnp.min(d, axis=1, keepdims=True)
        grp = jnp.min(jnp.where(d == dmin, iota_c, CLUSTERS),
                      axis=1, keepdims=True)
        onehot = jnp.where(iota_c == grp, 1.0, 0.0).astype(_f32) * maskc
        counts = jax.lax.dot_general(onehot, ones_col, (((0,), (0,)), ((), ())),
                                     preferred_element_type=_f32)
        sums = _mm_t_seq(onehot, q)
        new_cent = sums / jnp.maximum(counts, 1.0)
        cent = jnp.where(counts > 0, new_cent, cent)

    scale = _f32(1.0 / (HEAD_DIM ** 0.5))
    logits_t = jax.lax.dot_general(k, cent, (((1,), (1,)), ((), ())),
                                   preferred_element_type=_f32) * scale
    logits_t = jnp.where(maskc > 0.0, logits_t, _f32(-1e9))
    mx = jnp.max(logits_t, axis=0, keepdims=True)
    e = jnp.exp(logits_t - mx)
    a = e / jnp.sum(e, axis=0, keepdims=True)
    out_c = jax.lax.dot_general(a, v, (((0,), (0,)), ((), ())),
                                preferred_element_type=_f32)
    outc_ref[0, 0] = jnp.concatenate(
        [out_c, jnp.zeros((CLUSTERS, _TW - HEAD_DIM), _f32)], axis=1)
    gid_ref[0, 0] = grp + (b * HEADS + h) * CLUSTERS


def _proj_body(g_ref, maskc_ref, wo_ref, bo_ref, y_ref):
    g = g_ref[0]
    y = jnp.dot(g, wo_ref[...], preferred_element_type=_f32)
    y_ref[0] = y * maskc_ref[0] + bo_ref[...]


def _sc_gather(table, idx):
    mesh = plsc.VectorSubcoreMesh(core_axis_name="core",
                                  subcore_axis_name="subcore")

    @pl.kernel(out_type=jax.ShapeDtypeStruct((_NIDX, _TW), _f32),
               mesh=mesh)
    def gather_kernel(tab_hbm, i_hbm, o_hbm):
        def body(i_vmem, o_vmem):
            pltpu.sync_copy(tab_hbm.at[i_vmem.at[0]], o_vmem)

        pltpu.emit_pipeline(
            body,
            grid=(_NIDX // _GW,),
            in_specs=[pl.BlockSpec((1, _GW), index_map=lambda i: (0, i))],
            out_specs=[pl.BlockSpec((_GW, _TW), index_map=lambda i: (i, 0))],
            core_axis_name=("core", "subcore"),
            dimension_semantics=(pltpu.PARALLEL,),
        )(i_hbm, o_hbm)

    return gather_kernel(table, idx)


def kernel(X, attn_mask, length_mask, Wq, bq, Wk, bk, Wv, bv, Wo, bo):
    pos = jnp.arange(L, dtype=jnp.int32)
    maskf = (attn_mask & (pos[None, :] < length_mask[:, None])).astype(_f32)
    maskc = maskf.reshape(B, L, 1)
    init_idx = jnp.linspace(0, L - 1, CLUSTERS).astype(jnp.int32)
    xinit = X[:, init_idx, :]

    def _hm(wmat):
        return wmat.reshape(HIDDEN, HEADS, HEAD_DIM).transpose(1, 0, 2)
    wqkv3 = jnp.concatenate([_hm(Wq), _hm(Wk), _hm(Wv)], axis=2)
    bqkv3 = jnp.concatenate([bq.reshape(HEADS, 1, HEAD_DIM),
                             bk.reshape(HEADS, 1, HEAD_DIM),
                             bv.reshape(HEADS, 1, HEAD_DIM)], axis=2)
    wo_pad = jnp.pad(Wo.reshape(HEADS, HEAD_DIM, HIDDEN),
                     ((0, 0), (0, _TW - HEAD_DIM), (0, 0)))
    wo_big = wo_pad.reshape(HEADS * _TW, HIDDEN)

    out_c, gid = pl.pallas_call(
        _attn_body,
        grid=(B, HEADS),
        in_specs=[
            pl.BlockSpec((1, L, HIDDEN), lambda b, h: (b, 0, 0)),
            pl.BlockSpec((1, CLUSTERS, HIDDEN), lambda b, h: (b, 0, 0)),
            pl.BlockSpec((1, L, 1), lambda b, h: (b, 0, 0)),
            pl.BlockSpec((1, HIDDEN, 3 * HEAD_DIM), lambda b, h: (h, 0, 0)),
            pl.BlockSpec((1, 1, 3 * HEAD_DIM), lambda b, h: (h, 0, 0)),
        ],
        out_specs=[
            pl.BlockSpec((1, 1, CLUSTERS, _TW), lambda b, h: (b, h, 0, 0)),
            pl.BlockSpec((1, 1, L, 1), lambda b, h: (b, h, 0, 0)),
        ],
        out_shape=[
            jax.ShapeDtypeStruct((B, HEADS, CLUSTERS, _TW), _f32),
            jax.ShapeDtypeStruct((B, HEADS, L, 1), jnp.int32),
        ],
    )(X, xinit, maskc, wqkv3, bqkv3)

    table = out_c.reshape(_NROWS, _TW)
    idxp = gid.reshape(B, HEADS, L).transpose(0, 2, 1).reshape(1, _NIDX)
    gath = _sc_gather(table, idxp).reshape(B, L, HEADS * _TW)

    out = pl.pallas_call(
        _proj_body,
        grid=(B, L // _LT),
        in_specs=[
            pl.BlockSpec((1, _LT, HEADS * _TW), lambda b, t: (b, t, 0)),
            pl.BlockSpec((1, _LT, 1), lambda b, t: (b, t, 0)),
            pl.BlockSpec((HEADS * _TW, HIDDEN), lambda b, t: (0, 0)),
            pl.BlockSpec((1, HIDDEN), lambda b, t: (0, 0)),
        ],
        out_specs=pl.BlockSpec((1, _LT, HIDDEN), lambda b, t: (b, t, 0)),
        out_shape=jax.ShapeDtypeStruct((B, L, HIDDEN), _f32),
    )(gath, maskc, wo_big, bo.reshape(1, -1))
    return out
